# baseline (speedup 1.0000x reference)
import jax
import jax.numpy as jnp
from jax import lax
from jax.experimental import pallas as pl
from jax.experimental.pallas import tpu as pltpu
from jax.experimental.pallas import tpu_sc as plsc

_B, _S, _D = 64, 4096, 128
_NC, _NS = 2, 16
_NW = _NC * _NS
_ROWS = _B * _S
_BPW = _B // _NW
_CH = 256
_NCHUNK = _S // _CH
_NJ = _D // 16
_MAXP = 500.0
_MINP = 0.0


def _sc_body(x_hbm, w_hbm, sol_hbm, par_hbm, out_hbm, xbuf, ybuf, wbuf, solbuf, parbuf):
    cid = lax.axis_index("c")
    sid = lax.axis_index("s")
    wid = sid * _NC + cid
    pltpu.sync_copy(par_hbm, parbuf)
    pltpu.sync_copy(sol_hbm.at[wid], solbuf)
    pvec = parbuf[...]
    coef = pvec[0]
    scale = pvec[1]
    solvec = solbuf[...]

    for bi in range(_BPW):
        b = wid * _BPW + bi
        is_sol = solvec[bi] > 0.5

        def chunk_body(ci, carry, b=b, is_sol=is_sol):
            base = b * _S + ci * _CH
            pltpu.sync_copy(x_hbm.at[pl.ds(base, _CH)], xbuf)
            pltpu.sync_copy(w_hbm.at[pl.ds(base, _CH)], wbuf)

            @pl.when(is_sol)
            def _():
                def row_solar(r, c):
                    wv16 = wbuf[pl.ds((r // 16) * 16, 16)]
                    wv = lax.gather(
                        wv16, jnp.full((16, 1), r % 16, jnp.int32),
                        lax.GatherDimensionNumbers(
                            offset_dims=(), collapsed_slice_dims=(0,),
                            start_index_map=(0,)),
                        slice_sizes=(1,),
                        mode=lax.GatherScatterMode.PROMISE_IN_BOUNDS)
                    ssrd_norm = jnp.minimum(jnp.maximum(wv * scale, 0.01), 1.0)
                    af = coef * ssrd_norm
                    xa = [xbuf[r, pl.ds(j * 16, 16)] * af for j in range(_NJ)]
                    acc = xa[0]
                    mx = xa[0]
                    mn = xa[0]
                    for v in xa[1:]:
                        acc = acc + v
                        mx = jnp.maximum(mx, v)
                        mn = jnp.minimum(mn, v)
                    target = jnp.sum(acc)
                    rng = jnp.maximum(jnp.max(mx) - jnp.min(mn), 1.0)
                    lmin = -rng
                    lmax = rng
                    for _i in range(5):
                        lmid = (lmin + lmax) * 0.5
                        sacc = None
                        for j in range(_NJ):
                            y = jnp.minimum(jnp.maximum(xa[j] - lmid, _MINP), _MAXP)
                            sacc = y if sacc is None else sacc + y
                        total = jnp.sum(sacc)
                        diff = total - target
                        conv = jnp.abs(diff) < 0.1
                        lmin = jnp.where((total > target) & (~conv), lmid, lmin)
                        lmax = jnp.where((total <= target) & (~conv), lmid, lmax)
                    lam = (lmin + lmax) * 0.5
                    for j in range(_NJ):
                        ybuf[r, pl.ds(j * 16, 16)] = jnp.minimum(
                            jnp.maximum(xa[j] - lam, _MINP), _MAXP)
                    return c

                lax.fori_loop(0, _CH, row_solar, 0)

            @pl.when(jnp.logical_not(is_sol))
            def _():
                def row_relu(r, c):
                    for j in range(_NJ):
                        ybuf[r, pl.ds(j * 16, 16)] = jnp.maximum(
                            xbuf[r, pl.ds(j * 16, 16)], 0.0)
                    return c

                lax.fori_loop(0, _CH, row_relu, 0)

            pltpu.sync_copy(ybuf, out_hbm.at[pl.ds(base, _CH)])
            return carry

        lax.fori_loop(0, _NCHUNK, chunk_body, 0)


def kernel(x, weather_data, is_solar, unit_ids, c_prime, alpha, alpha_prime, ssrd_scale, A, eta):
    del unit_ids
    coef = c_prime * A * eta / (alpha + alpha_prime) * _MAXP
    params = jnp.zeros((16,), jnp.float32)
    params = params.at[0].set(coef).at[1].set(ssrd_scale.astype(jnp.float32))
    solf = (is_solar[:, 0] == 1).astype(jnp.float32).reshape(_NW, _BPW)
    solf = jnp.pad(solf, ((0, 0), (0, 16 - _BPW)))
    xf = x.reshape(_ROWS, _D)
    wf = weather_data.reshape(_ROWS)
    mesh = plsc.VectorSubcoreMesh(core_axis_name="c", subcore_axis_name="s")
    out = pl.kernel(
        _sc_body,
        out_type=jax.ShapeDtypeStruct((_ROWS, _D), jnp.float32),
        mesh=mesh,
        compiler_params=pltpu.CompilerParams(needs_layout_passes=False),
        scratch_types=[
            pltpu.VMEM((_CH, _D), jnp.float32),
            pltpu.VMEM((_CH, _D), jnp.float32),
            pltpu.VMEM((_CH,), jnp.float32),
            pltpu.VMEM((16,), jnp.float32),
            pltpu.VMEM((16,), jnp.float32),
        ],
    )(xf, wf, solf, params)
    return out.reshape(_B, _S, _D)


# speedup vs baseline: 2.2659x; 2.2659x over previous
import jax
import jax.numpy as jnp
from jax import lax
from jax.experimental import pallas as pl
from jax.experimental.pallas import tpu as pltpu
from jax.experimental.pallas import tpu_sc as plsc

_B, _S, _D = 64, 4096, 128
_NC, _NS = 2, 16
_NW = _NC * _NS
_ROWS = _B * _S
_CH = 128
_CPB = _S // _CH
_NK = _B
_NJ = _D // 16
_MAXP = 500.0
_MINP = 0.0


def _splat(vec16, lane):
    return lax.gather(
        vec16, jnp.full((16, 1), lane, jnp.int32),
        lax.GatherDimensionNumbers(
            offset_dims=(), collapsed_slice_dims=(0,), start_index_map=(0,)),
        slice_sizes=(1,),
        mode=lax.GatherScatterMode.PROMISE_IN_BOUNDS)


def _sc_body(x_hbm, w_hbm, sol_hbm, par_hbm, out_hbm,
             xb0, xb1, yb0, yb1, wb0, wb1, afb, solbuf, parbuf,
             sin0, sin1, sout0, sout1):
    cid = lax.axis_index("c")
    sid = lax.axis_index("s")
    wid = sid * _NC + cid
    pltpu.sync_copy(par_hbm, parbuf)
    pltpu.sync_copy(sol_hbm, solbuf)
    pvec = parbuf[...]
    coef = pvec[0]
    scale = pvec[1]

    xbufs, ybufs, wbufs = (xb0, xb1), (yb0, yb1), (wb0, wb1)
    sins, souts = (sin0, sin1), (sout0, sout1)

    def chunk_base(k):
        return (k * _CPB + wid) * _CH

    def issue_in(k, b):
        base = chunk_base(k)
        pltpu.async_copy(x_hbm.at[pl.ds(base, _CH)], xbufs[b], sins[b])
        pltpu.async_copy(w_hbm.at[pl.ds(base, _CH)], wbufs[b], sins[b])

    issue_in(0, 0)

    def outer(ko, carry):
        for bsel in (0, 1):
            k = ko * 2 + bsel
            xb, yb, wb = xbufs[bsel], ybufs[bsel], wbufs[bsel]
            pltpu.make_async_copy(x_hbm.at[pl.ds(0, _CH)], xb, sins[bsel]).wait()
            pltpu.make_async_copy(w_hbm.at[pl.ds(0, _CH)], wb, sins[bsel]).wait()

            @pl.when(k + 1 < _NK)
            def _():
                issue_in(k + 1, bsel ^ 1)

            s16 = solbuf[pl.ds((k // 16) * 16, 16)]
            is_sol = jnp.max(_splat(s16, k % 16)) > 0.5

            @pl.when(k >= 2)
            def _():
                pltpu.make_async_copy(yb, out_hbm.at[pl.ds(0, _CH)],
                                      souts[bsel]).wait()

            @pl.when(is_sol)
            def _():
                for i in range(_CH // 16):
                    wv = wb[pl.ds(i * 16, 16)]
                    afb[pl.ds(i * 16, 16)] = coef * jnp.minimum(
                        jnp.maximum(wv * scale, 0.01), 1.0)

                @plsc.parallel_loop(0, _CH, 1, unroll=4)
                def row_solar(r):
                    af16 = afb[pl.ds((r // 16) * 16, 16)]
                    af = _splat(af16, r % 16)
                    xa = [xb[r, pl.ds(j * 16, 16)] * af for j in range(_NJ)]
                    acc = xa[0]
                    mx = xa[0]
                    mn = xa[0]
                    for v in xa[1:]:
                        acc = acc + v
                        mx = jnp.maximum(mx, v)
                        mn = jnp.minimum(mn, v)
                    target = jnp.sum(acc)
                    rng = jnp.maximum(jnp.max(mx) - jnp.min(mn), 1.0)
                    lmin = -rng
                    lmax = rng
                    for _i in range(5):
                        lmid = (lmin + lmax) * 0.5
                        sacc = None
                        for j in range(_NJ):
                            y = jnp.minimum(jnp.maximum(xa[j] - lmid, _MINP), _MAXP)
                            sacc = y if sacc is None else sacc + y
                        total = jnp.sum(sacc)
                        diff = total - target
                        conv = jnp.abs(diff) < 0.1
                        lmin = jnp.where((total > target) & (~conv), lmid, lmin)
                        lmax = jnp.where((total <= target) & (~conv), lmid, lmax)
                    lam = (lmin + lmax) * 0.5
                    for j in range(_NJ):
                        yb[r, pl.ds(j * 16, 16)] = jnp.minimum(
                            jnp.maximum(xa[j] - lam, _MINP), _MAXP)

            @pl.when(jnp.logical_not(is_sol))
            def _():
                @plsc.parallel_loop(0, _CH, 1, unroll=4)
                def row_relu(r):
                    for j in range(_NJ):
                        yb[r, pl.ds(j * 16, 16)] = jnp.maximum(
                            xb[r, pl.ds(j * 16, 16)], 0.0)

            pltpu.async_copy(yb, out_hbm.at[pl.ds(chunk_base(k), _CH)],
                             souts[bsel])
        return carry

    lax.fori_loop(0, _NK // 2, outer, 0)
    pltpu.make_async_copy(yb0, out_hbm.at[pl.ds(0, _CH)], souts[0]).wait()
    pltpu.make_async_copy(yb1, out_hbm.at[pl.ds(0, _CH)], souts[1]).wait()


def kernel(x, weather_data, is_solar, unit_ids, c_prime, alpha, alpha_prime, ssrd_scale, A, eta):
    del unit_ids
    coef = c_prime * A * eta / (alpha + alpha_prime) * _MAXP
    params = jnp.zeros((16,), jnp.float32)
    params = params.at[0].set(coef).at[1].set(ssrd_scale.astype(jnp.float32))
    solf = (is_solar[:, 0] == 1).astype(jnp.float32)
    xf = x.reshape(_ROWS, _D)
    wf = weather_data.reshape(_ROWS)
    mesh = plsc.VectorSubcoreMesh(core_axis_name="c", subcore_axis_name="s")
    out = pl.kernel(
        _sc_body,
        out_type=jax.ShapeDtypeStruct((_ROWS, _D), jnp.float32),
        mesh=mesh,
        compiler_params=pltpu.CompilerParams(needs_layout_passes=False),
        scratch_types=[
            pltpu.VMEM((_CH, _D), jnp.float32),
            pltpu.VMEM((_CH, _D), jnp.float32),
            pltpu.VMEM((_CH, _D), jnp.float32),
            pltpu.VMEM((_CH, _D), jnp.float32),
            pltpu.VMEM((_CH,), jnp.float32),
            pltpu.VMEM((_CH,), jnp.float32),
            pltpu.VMEM((_CH,), jnp.float32),
            pltpu.VMEM((_B,), jnp.float32),
            pltpu.VMEM((16,), jnp.float32),
            pltpu.SemaphoreType.DMA,
            pltpu.SemaphoreType.DMA,
            pltpu.SemaphoreType.DMA,
            pltpu.SemaphoreType.DMA,
        ],
    )(xf, wf, solf, params)
    return out.reshape(_B, _S, _D)
